# baseline (speedup 1.0000x reference)
import jax
import jax.numpy as jnp
from jax import lax
from jax.experimental import pallas as pl
from jax.experimental.pallas import tpu as pltpu
from jax.experimental.pallas import tpu_sc as plsc

N_NODES = 10000
N_EDGES = 320000
EH = N_EDGES // 2
D = 128
DW = D // 2
DE = 16
LANES = 16

NC = 2
NS = 16
NW = NC * NS
EPR = EH // NW
CH = 40
NCH = EPR // CH

_SC_MESH = plsc.VectorSubcoreMesh(
    core_axis_name="c", subcore_axis_name="s", num_cores=NC, num_subcores=NS)


def _pack_bf16_halves(y):
    yb = y.astype(jnp.bfloat16)
    lo = lax.bitcast_convert_type(yb[:, :DW], jnp.uint16).astype(jnp.uint32)
    hi = lax.bitcast_convert_type(yb[:, DW:], jnp.uint16).astype(jnp.uint32)
    return lax.bitcast_convert_type(lo | (hi << 16), jnp.int32)


def _tables_body(x_ref, ws_ref, wr_ref, xs_ref, xr_ref):
    xs_ref[...] = _pack_bf16_halves(
        jnp.dot(x_ref[...], ws_ref[...], preferred_element_type=jnp.float32))
    xr_ref[...] = _pack_bf16_halves(
        jnp.dot(x_ref[...], wr_ref[...], preferred_element_type=jnp.float32))


def _combine_body(g_ref, ea_ref, we_ref, b_ref, o_ref):
    mm = (jnp.dot(ea_ref[...], we_ref[...], preferred_element_type=jnp.float32)
          + b_ref[...])
    g = g_ref[...]

    @pl.when(pl.program_id(1) == 0)
    def _():
        o_ref[...] = lax.bitcast_convert_type(
            lax.shift_left(g, 16), jnp.float32) + mm

    @pl.when(pl.program_id(1) == 1)
    def _():
        o_ref[...] = lax.bitcast_convert_type(
            g & jnp.int32(-65536), jnp.float32) + mm


def _gather_sum_body(xs_hbm, xr_hbm, sidx_hbm, ridx_hbm, out_hbm,
                     sidx_v, ridx_v,
                     al0, al1, bl0, bl1, o0, o1,
                     sg0, sg1, so0, so1):
    wid = lax.axis_index("s") * NC + lax.axis_index("c")
    base = wid * EPR
    pltpu.sync_copy(sidx_hbm.at[0, wid], sidx_v.at[:, pl.ds(0, CH)])
    pltpu.sync_copy(sidx_hbm.at[1, wid], sidx_v.at[:, pl.ds(CH, CH)])
    pltpu.sync_copy(ridx_hbm.at[0, wid], ridx_v.at[:, pl.ds(0, CH)])
    pltpu.sync_copy(ridx_hbm.at[1, wid], ridx_v.at[:, pl.ds(CH, CH)])

    a = (al0, al1)
    b = (bl0, bl1)
    o = (o0, o1)
    sg = (sg0, sg1)
    so = (so0, so1)

    def gather(c, k):
        pltpu.async_copy(xs_hbm.at[sidx_v.at[c]], a[k], sg[k])
        pltpu.async_copy(xr_hbm.at[ridx_v.at[c]], b[k], sg[k])

    def wait_gather(k):
        pltpu.make_async_copy(xs_hbm.at[pl.ds(0, 2 * CH)], a[k], sg[k]).wait()
        pltpu.make_async_copy(xr_hbm.at[pl.ds(0, 2 * CH)], b[k], sg[k]).wait()

    def wait_write(k):
        pltpu.make_async_copy(o[k], out_hbm.at[pl.ds(0, CH)], so[k]).wait()

    gather(0, 0)
    gather(1, 1)

    def process(c, k):
        wait_gather(k)

        @pl.when(c >= 2)
        def _():
            wait_write(k)

        def row_body(i, carry2):
            for j in range(DW // LANES):
                sl = pl.ds(j * LANES, LANES)
                a0 = plsc.unpack(plsc.bitcast(a[k][i, sl], jnp.bfloat16),
                                 format=plsc.PackFormat.INTERLEAVED)
                b0 = plsc.unpack(plsc.bitcast(b[k][i, sl], jnp.bfloat16),
                                 format=plsc.PackFormat.INTERLEAVED)
                a1 = plsc.unpack(plsc.bitcast(a[k][CH + i, sl], jnp.bfloat16),
                                 format=plsc.PackFormat.INTERLEAVED)
                b1 = plsc.unpack(plsc.bitcast(b[k][CH + i, sl], jnp.bfloat16),
                                 format=plsc.PackFormat.INTERLEAVED)
                lo0 = a0[0] + b0[0]
                hi0 = a0[1] + b0[1]
                lo1 = a1[0] + b1[0]
                hi1 = a1[1] + b1[1]
                wlo = plsc.bitcast(
                    plsc.pack(lo0, lo1, format=plsc.PackFormat.INTERLEAVED),
                    jnp.int32)
                whi = plsc.bitcast(
                    plsc.pack(hi0, hi1, format=plsc.PackFormat.INTERLEAVED),
                    jnp.int32)
                o[k][i, sl] = wlo
                o[k][i, pl.ds(DW + j * LANES, LANES)] = whi
            return carry2

        lax.fori_loop(0, CH, row_body, 0, unroll=4)

        pltpu.async_copy(o[k], out_hbm.at[pl.ds(base + c * CH, CH)], so[k])

        @pl.when(c + 2 < NCH)
        def _():
            gather(c + 2, k)

    def pair_body(p, carry):
        process(p * 2, 0)
        process(p * 2 + 1, 1)
        return carry

    lax.fori_loop(0, NCH // 2, pair_body, 0)
    process(jnp.int32(NCH - 1), 0)
    wait_write(0)
    wait_write(1)


_gather_sum = pl.kernel(
    _gather_sum_body,
    out_type=jax.ShapeDtypeStruct((EH, D), jnp.int32),
    mesh=_SC_MESH,
    compiler_params=pltpu.CompilerParams(use_tc_tiling_on_sc=False,
                                         needs_layout_passes=False),
    scratch_types=[
        pltpu.VMEM((NCH, 2 * CH), jnp.int32),
        pltpu.VMEM((NCH, 2 * CH), jnp.int32),
        pltpu.VMEM((2 * CH, DW), jnp.int32),
        pltpu.VMEM((2 * CH, DW), jnp.int32),
        pltpu.VMEM((2 * CH, DW), jnp.int32),
        pltpu.VMEM((2 * CH, DW), jnp.int32),
        pltpu.VMEM((CH, D), jnp.int32),
        pltpu.VMEM((CH, D), jnp.int32),
        pltpu.SemaphoreType.DMA,
        pltpu.SemaphoreType.DMA,
        pltpu.SemaphoreType.DMA,
        pltpu.SemaphoreType.DMA,
    ],
)

_EBH = 1600
_NBH = EH // _EBH


def kernel(x, edge_index, edge_attr, W, b):
    s_idx = edge_index[0].astype(jnp.int32).reshape(2, NW, NCH, CH)
    r_idx = edge_index[1].astype(jnp.int32).reshape(2, NW, NCH, CH)
    ws = W[:D]
    wr = W[D:2 * D]
    we = W[2 * D:]
    b2 = b.reshape(1, D)

    xs, xr = pl.pallas_call(
        _tables_body,
        out_shape=[jax.ShapeDtypeStruct((N_NODES, DW), jnp.int32)] * 2,
    )(x, ws, wr)

    g = _gather_sum(xs, xr, s_idx, r_idx)

    out = pl.pallas_call(
        _combine_body,
        grid=(_NBH, 2),
        in_specs=[
            pl.BlockSpec((_EBH, D), lambda i, j: (i, 0)),
            pl.BlockSpec((_EBH, DE), lambda i, j: (i + j * _NBH, 0)),
            pl.BlockSpec((DE, D), lambda i, j: (0, 0)),
            pl.BlockSpec((1, D), lambda i, j: (0, 0)),
        ],
        out_specs=pl.BlockSpec((_EBH, D), lambda i, j: (i + j * _NBH, 0)),
        out_shape=jax.ShapeDtypeStruct((N_EDGES, D), jnp.float32),
    )(g, edge_attr, we, b2)

    return (x, edge_index, out)


# speedup vs baseline: 1.1509x; 1.1509x over previous
import jax
import jax.numpy as jnp
from jax import lax
from jax.experimental import pallas as pl
from jax.experimental.pallas import tpu as pltpu
from jax.experimental.pallas import tpu_sc as plsc

N_NODES = 10000
N_EDGES = 320000
EH = N_EDGES // 2
D = 128
DW = D // 2
DE = 16
LANES = 16

NC = 2
NS = 16
NW = NC * NS
EPR = EH // NW
CH = 40
NCH = EPR // CH

_SC_MESH = plsc.VectorSubcoreMesh(
    core_axis_name="c", subcore_axis_name="s", num_cores=NC, num_subcores=NS)


def _pack_bf16_halves(y):
    yb = y.astype(jnp.bfloat16)
    lo = lax.bitcast_convert_type(yb[:, :DW], jnp.uint16).astype(jnp.uint32)
    hi = lax.bitcast_convert_type(yb[:, DW:], jnp.uint16).astype(jnp.uint32)
    return lax.bitcast_convert_type(lo | (hi << 16), jnp.int32)


def _tables_body(x_ref, ws_ref, wr_ref, xs_ref, xr_ref):
    xs_ref[...] = _pack_bf16_halves(
        jnp.dot(x_ref[...], ws_ref[...], preferred_element_type=jnp.float32))
    xr_ref[...] = _pack_bf16_halves(
        jnp.dot(x_ref[...], wr_ref[...], preferred_element_type=jnp.float32))


def _combine_body(g_ref, ea_ref, we_ref, b_ref, o_ref):
    mm = (jnp.dot(ea_ref[...], we_ref[...], preferred_element_type=jnp.float32)
          + b_ref[...])
    g = g_ref[...]

    @pl.when(pl.program_id(1) == 0)
    def _():
        o_ref[...] = lax.bitcast_convert_type(
            lax.shift_left(g, 16), jnp.float32) + mm

    @pl.when(pl.program_id(1) == 1)
    def _():
        o_ref[...] = lax.bitcast_convert_type(
            g & jnp.int32(-65536), jnp.float32) + mm


def _gather_sum_body(xs_hbm, xr_hbm, sidx_hbm, ridx_hbm, out_hbm,
                     sidx_v, ridx_v,
                     al0, al1, bl0, bl1, o0, o1,
                     sg0, sg1, so0, so1):
    wid = lax.axis_index("s") * NC + lax.axis_index("c")
    base = wid * EPR
    pltpu.sync_copy(sidx_hbm.at[0, wid], sidx_v.at[:, pl.ds(0, CH)])
    pltpu.sync_copy(sidx_hbm.at[1, wid], sidx_v.at[:, pl.ds(CH, CH)])
    pltpu.sync_copy(ridx_hbm.at[0, wid], ridx_v.at[:, pl.ds(0, CH)])
    pltpu.sync_copy(ridx_hbm.at[1, wid], ridx_v.at[:, pl.ds(CH, CH)])

    a = (al0, al1)
    b = (bl0, bl1)
    o = (o0, o1)
    sg = (sg0, sg1)
    so = (so0, so1)

    def gather(c, k):
        pltpu.async_copy(xs_hbm.at[sidx_v.at[c]], a[k], sg[k])
        pltpu.async_copy(xr_hbm.at[ridx_v.at[c]], b[k], sg[k])

    def wait_gather(k):
        pltpu.make_async_copy(xs_hbm.at[pl.ds(0, 2 * CH)], a[k], sg[k]).wait()
        pltpu.make_async_copy(xr_hbm.at[pl.ds(0, 2 * CH)], b[k], sg[k]).wait()

    def wait_write(k):
        pltpu.make_async_copy(o[k], out_hbm.at[pl.ds(0, CH)], so[k]).wait()

    gather(0, 0)
    gather(1, 1)

    def process(c, k):
        wait_gather(k)

        @pl.when(c >= 2)
        def _():
            wait_write(k)

        def row_body(i, carry2):
            for j in range(DW // LANES):
                sl = pl.ds(j * LANES, LANES)
                a0 = plsc.unpack(plsc.bitcast(a[k][i, sl], jnp.bfloat16),
                                 format=plsc.PackFormat.INTERLEAVED)
                b0 = plsc.unpack(plsc.bitcast(b[k][i, sl], jnp.bfloat16),
                                 format=plsc.PackFormat.INTERLEAVED)
                a1 = plsc.unpack(plsc.bitcast(a[k][CH + i, sl], jnp.bfloat16),
                                 format=plsc.PackFormat.INTERLEAVED)
                b1 = plsc.unpack(plsc.bitcast(b[k][CH + i, sl], jnp.bfloat16),
                                 format=plsc.PackFormat.INTERLEAVED)
                lo0 = a0[0] + b0[0]
                hi0 = a0[1] + b0[1]
                lo1 = a1[0] + b1[0]
                hi1 = a1[1] + b1[1]
                wlo = plsc.bitcast(
                    plsc.pack(lo0, lo1, format=plsc.PackFormat.INTERLEAVED),
                    jnp.int32)
                whi = plsc.bitcast(
                    plsc.pack(hi0, hi1, format=plsc.PackFormat.INTERLEAVED),
                    jnp.int32)
                o[k][i, sl] = wlo
                o[k][i, pl.ds(DW + j * LANES, LANES)] = whi
            return carry2

        lax.fori_loop(0, CH, row_body, 0, unroll=4)

        pltpu.async_copy(o[k], out_hbm.at[pl.ds(base + c * CH, CH)], so[k])

        @pl.when(c + 2 < NCH)
        def _():
            gather(c + 2, k)

    def pair_body(p, carry):
        process(p * 2, 0)
        process(p * 2 + 1, 1)
        return carry

    lax.fori_loop(0, NCH // 2, pair_body, 0)
    process(jnp.int32(NCH - 1), 0)
    wait_write(0)
    wait_write(1)


_gather_sum = pl.kernel(
    _gather_sum_body,
    out_type=jax.ShapeDtypeStruct((EH, D), jnp.int32),
    mesh=_SC_MESH,
    compiler_params=pltpu.CompilerParams(use_tc_tiling_on_sc=False,
                                         needs_layout_passes=False),
    scratch_types=[
        pltpu.VMEM((NCH, 2 * CH), jnp.int32),
        pltpu.VMEM((NCH, 2 * CH), jnp.int32),
        pltpu.VMEM((2 * CH, DW), jnp.int32),
        pltpu.VMEM((2 * CH, DW), jnp.int32),
        pltpu.VMEM((2 * CH, DW), jnp.int32),
        pltpu.VMEM((2 * CH, DW), jnp.int32),
        pltpu.VMEM((CH, D), jnp.int32),
        pltpu.VMEM((CH, D), jnp.int32),
        pltpu.SemaphoreType.DMA,
        pltpu.SemaphoreType.DMA,
        pltpu.SemaphoreType.DMA,
        pltpu.SemaphoreType.DMA,
    ],
)

_EBH = 3200
_NBH = EH // _EBH


def kernel(x, edge_index, edge_attr, W, b):
    s_idx = edge_index[0].astype(jnp.int32).reshape(2, NW, NCH, CH)
    r_idx = edge_index[1].astype(jnp.int32).reshape(2, NW, NCH, CH)
    ws = W[:D]
    wr = W[D:2 * D]
    we = W[2 * D:]
    b2 = b.reshape(1, D)

    xs, xr = pl.pallas_call(
        _tables_body,
        out_shape=[jax.ShapeDtypeStruct((N_NODES, DW), jnp.int32)] * 2,
    )(x, ws, wr)

    g = _gather_sum(xs, xr, s_idx, r_idx)

    out = pl.pallas_call(
        _combine_body,
        grid=(_NBH, 2),
        in_specs=[
            pl.BlockSpec((_EBH, D), lambda i, j: (i, 0)),
            pl.BlockSpec((_EBH, DE), lambda i, j: (i + j * _NBH, 0)),
            pl.BlockSpec((DE, D), lambda i, j: (0, 0)),
            pl.BlockSpec((1, D), lambda i, j: (0, 0)),
        ],
        out_specs=pl.BlockSpec((_EBH, D), lambda i, j: (i + j * _NBH, 0)),
        out_shape=jax.ShapeDtypeStruct((N_EDGES, D), jnp.float32),
    )(g, edge_attr, we, b2)

    return (x, edge_index, out)


# speedup vs baseline: 1.2247x; 1.0641x over previous
import jax
import jax.numpy as jnp
from jax import lax
from jax.experimental import pallas as pl
from jax.experimental.pallas import tpu as pltpu
from jax.experimental.pallas import tpu_sc as plsc

N_NODES = 10000
N_EDGES = 320000
EH = N_EDGES // 2
D = 128
DW = D // 2
DE = 16
LANES = 16

NC = 2
NS = 16
NW = NC * NS
EPR = EH // NW
CH = 40
NCH = EPR // CH

_SC_MESH = plsc.VectorSubcoreMesh(
    core_axis_name="c", subcore_axis_name="s", num_cores=NC, num_subcores=NS)


def _pack_bf16_halves(y):
    yb = y.astype(jnp.bfloat16)
    lo = lax.bitcast_convert_type(yb[:, :DW], jnp.uint16).astype(jnp.uint32)
    hi = lax.bitcast_convert_type(yb[:, DW:], jnp.uint16).astype(jnp.uint32)
    return lax.bitcast_convert_type(lo | (hi << 16), jnp.int32)


def _tables_body(x_ref, ws_ref, wr_ref, xs_ref, xr_ref):
    xs_ref[...] = _pack_bf16_halves(
        jnp.dot(x_ref[...], ws_ref[...], preferred_element_type=jnp.float32))
    xr_ref[...] = _pack_bf16_halves(
        jnp.dot(x_ref[...], wr_ref[...], preferred_element_type=jnp.float32))


def _combine_body(g_ref, ea_ref, we_ref, b_ref, o_ref):
    mm = (jnp.dot(ea_ref[...], we_ref[...], preferred_element_type=jnp.float32)
          + b_ref[...])
    g = g_ref[...]

    @pl.when(pl.program_id(1) == 0)
    def _():
        o_ref[...] = lax.bitcast_convert_type(
            lax.shift_left(g, 16), jnp.float32) + mm

    @pl.when(pl.program_id(1) == 1)
    def _():
        o_ref[...] = lax.bitcast_convert_type(
            g & jnp.int32(-65536), jnp.float32) + mm


def _gather_sum_body(xs_hbm, xr_hbm, sidx_hbm, ridx_hbm, out_hbm,
                     sidx_v, ridx_v,
                     al0, al1, bl0, bl1, o0, o1,
                     sg0, sg1, so0, so1):
    wid = lax.axis_index("s") * NC + lax.axis_index("c")
    base = wid * EPR
    pltpu.sync_copy(sidx_hbm.at[0, wid], sidx_v.at[:, pl.ds(0, CH)])
    pltpu.sync_copy(sidx_hbm.at[1, wid], sidx_v.at[:, pl.ds(CH, CH)])
    pltpu.sync_copy(ridx_hbm.at[0, wid], ridx_v.at[:, pl.ds(0, CH)])
    pltpu.sync_copy(ridx_hbm.at[1, wid], ridx_v.at[:, pl.ds(CH, CH)])

    a = (al0, al1)
    b = (bl0, bl1)
    o = (o0, o1)
    sg = (sg0, sg1)
    so = (so0, so1)

    def gather(c, k):
        pltpu.async_copy(xs_hbm.at[sidx_v.at[c]], a[k], sg[k])
        pltpu.async_copy(xr_hbm.at[ridx_v.at[c]], b[k], sg[k])

    def wait_gather(k):
        pltpu.make_async_copy(xs_hbm.at[pl.ds(0, 2 * CH)], a[k], sg[k]).wait()
        pltpu.make_async_copy(xr_hbm.at[pl.ds(0, 2 * CH)], b[k], sg[k]).wait()

    def wait_write(k):
        pltpu.make_async_copy(o[k], out_hbm.at[pl.ds(0, CH)], so[k]).wait()

    gather(0, 0)
    gather(1, 1)

    def process(c, k):
        wait_gather(k)

        @pl.when(c >= 2)
        def _():
            wait_write(k)

        def row_body(i, carry2):
            for j in range(DW // LANES):
                sl = pl.ds(j * LANES, LANES)
                a0 = plsc.unpack(plsc.bitcast(a[k][i, sl], jnp.bfloat16),
                                 format=plsc.PackFormat.INTERLEAVED)
                b0 = plsc.unpack(plsc.bitcast(b[k][i, sl], jnp.bfloat16),
                                 format=plsc.PackFormat.INTERLEAVED)
                a1 = plsc.unpack(plsc.bitcast(a[k][CH + i, sl], jnp.bfloat16),
                                 format=plsc.PackFormat.INTERLEAVED)
                b1 = plsc.unpack(plsc.bitcast(b[k][CH + i, sl], jnp.bfloat16),
                                 format=plsc.PackFormat.INTERLEAVED)
                lo0 = a0[0] + b0[0]
                hi0 = a0[1] + b0[1]
                lo1 = a1[0] + b1[0]
                hi1 = a1[1] + b1[1]
                wlo = plsc.bitcast(
                    plsc.pack(lo0, lo1, format=plsc.PackFormat.INTERLEAVED),
                    jnp.int32)
                whi = plsc.bitcast(
                    plsc.pack(hi0, hi1, format=plsc.PackFormat.INTERLEAVED),
                    jnp.int32)
                o[k][i, sl] = wlo
                o[k][i, pl.ds(DW + j * LANES, LANES)] = whi
            return carry2

        lax.fori_loop(0, CH, row_body, 0, unroll=4)

        pltpu.async_copy(o[k], out_hbm.at[pl.ds(base + c * CH, CH)], so[k])

        @pl.when(c + 2 < NCH)
        def _():
            gather(c + 2, k)

    def pair_body(p, carry):
        process(p * 2, 0)
        process(p * 2 + 1, 1)
        return carry

    lax.fori_loop(0, NCH // 2, pair_body, 0)
    process(jnp.int32(NCH - 1), 0)
    wait_write(0)
    wait_write(1)


_gather_sum = pl.kernel(
    _gather_sum_body,
    out_type=jax.ShapeDtypeStruct((EH, D), jnp.int32),
    mesh=_SC_MESH,
    compiler_params=pltpu.CompilerParams(use_tc_tiling_on_sc=False,
                                         needs_layout_passes=False),
    scratch_types=[
        pltpu.VMEM((NCH, 2 * CH), jnp.int32),
        pltpu.VMEM((NCH, 2 * CH), jnp.int32),
        pltpu.VMEM((2 * CH, DW), jnp.int32),
        pltpu.VMEM((2 * CH, DW), jnp.int32),
        pltpu.VMEM((2 * CH, DW), jnp.int32),
        pltpu.VMEM((2 * CH, DW), jnp.int32),
        pltpu.VMEM((CH, D), jnp.int32),
        pltpu.VMEM((CH, D), jnp.int32),
        pltpu.SemaphoreType.DMA,
        pltpu.SemaphoreType.DMA,
        pltpu.SemaphoreType.DMA,
        pltpu.SemaphoreType.DMA,
    ],
)

_EBH = 6400
_NBH = EH // _EBH


def kernel(x, edge_index, edge_attr, W, b):
    s_idx = edge_index[0].astype(jnp.int32).reshape(2, NW, NCH, CH)
    r_idx = edge_index[1].astype(jnp.int32).reshape(2, NW, NCH, CH)
    ws = W[:D]
    wr = W[D:2 * D]
    we = W[2 * D:]
    b2 = b.reshape(1, D)

    xs, xr = pl.pallas_call(
        _tables_body,
        out_shape=[jax.ShapeDtypeStruct((N_NODES, DW), jnp.int32)] * 2,
    )(x, ws, wr)

    g = _gather_sum(xs, xr, s_idx, r_idx)

    out = pl.pallas_call(
        _combine_body,
        grid=(_NBH, 2),
        in_specs=[
            pl.BlockSpec((_EBH, D), lambda i, j: (i, 0)),
            pl.BlockSpec((_EBH, DE), lambda i, j: (i + j * _NBH, 0)),
            pl.BlockSpec((DE, D), lambda i, j: (0, 0)),
            pl.BlockSpec((1, D), lambda i, j: (0, 0)),
        ],
        out_specs=pl.BlockSpec((_EBH, D), lambda i, j: (i + j * _NBH, 0)),
        out_shape=jax.ShapeDtypeStruct((N_EDGES, D), jnp.float32),
    )(g, edge_attr, we, b2)

    return (x, edge_index, out)
